# baseline (speedup 1.0000x reference)
import functools

import jax
import jax.numpy as jnp
from jax import lax
from jax.experimental import pallas as pl
from jax.experimental.pallas import tpu as pltpu
from jax.experimental.pallas import tpu_sc as plsc

_C = 128
_L = 16


def kernel(heads, rels, entity_table, rel_table):
    B = heads.shape[0]
    E, D = entity_table.shape
    R = rel_table.shape[0]

    info = plsc.get_sparse_core_info()
    NW = info.num_cores * info.num_subcores
    b_w = B // NW
    n_ch = b_w // _C
    n_grp = _C // _L
    assert b_w * NW == B and n_ch * _C == b_w and B % (8 * NW) == 0

    et4 = entity_table.reshape(E // 4, 128)
    rt4 = rel_table.reshape(R // 4, 128)
    hq = lax.shift_right_logical(heads, 2)
    rq = lax.shift_right_logical(rels, 2)

    mesh = plsc.VectorSubcoreMesh(core_axis_name="c", subcore_axis_name="s")

    @functools.partial(
        pl.kernel,
        mesh=mesh,
        compiler_params=pltpu.CompilerParams(needs_layout_passes=False),
        out_type=(
            jax.ShapeDtypeStruct((B * D,), jnp.float32),
            jax.ShapeDtypeStruct((B * D,), jnp.float32),
        ),
        scratch_types=[
            pltpu.VMEM((b_w,), jnp.int32),
            pltpu.VMEM((b_w,), jnp.int32),
            pltpu.VMEM((2 * n_ch, _C), jnp.int32),
            pltpu.VMEM((2, _C, 128), jnp.float32),
            pltpu.VMEM((b_w * D,), jnp.float32),
            pltpu.VMEM((b_w * D,), jnp.float32),
            pltpu.SemaphoreType.DMA,
            pltpu.SemaphoreType.DMA,
        ],
    )
    def _gather2(heads_hbm, rels_hbm, hq_hbm, rq_hbm, et4_hbm, rt4_hbm,
                 out_h_hbm, out_r_hbm,
                 hidx_v, ridx_v, qidx_v, lines, houtv, routv, sem_a, sem_b):
        wid = lax.axis_index("s") * info.num_cores + lax.axis_index("c")
        base = wid * b_w
        iota = lax.iota(jnp.int32, _L)
        sems = (sem_a, sem_b)

        pltpu.sync_copy(heads_hbm.at[pl.ds(base, b_w)], hidx_v)
        pltpu.sync_copy(rels_hbm.at[pl.ds(base, b_w)], ridx_v)
        for c in range(n_ch):
            pltpu.sync_copy(hq_hbm.at[pl.ds(base + c * _C, _C)], qidx_v.at[c])
            pltpu.sync_copy(rq_hbm.at[pl.ds(base + c * _C, _C)],
                            qidx_v.at[n_ch + c])

        def fire(stream_tbl, qslot, buf):
            return pltpu.async_copy(
                stream_tbl.at[qidx_v.at[qslot]], lines.at[buf], sems[buf])

        def extract(buf, idx_src, off, outv):
            def grp(g, _):
                hv = idx_src[pl.ds(off + g * _L, _L)]
                qv = (hv & 3) * D
                for lane in range(_L):
                    rowv = iota * 0 + (g * _L + lane)
                    a0 = iota + qv[lane]
                    v0 = plsc.load_gather(lines, [rowv * 0 + buf, rowv, a0])
                    v1 = plsc.load_gather(lines,
                                          [rowv * 0 + buf, rowv, a0 + _L])
                    ob = (off + g * _L + lane) * D
                    outv[pl.ds(ob, _L)] = v0
                    outv[pl.ds(ob + _L, _L)] = v1
                return 0
            lax.fori_loop(0, n_grp, grp, 0)

        jobs = ([(et4_hbm, c, hidx_v, c * _C, houtv) for c in range(n_ch)] +
                [(rt4_hbm, n_ch + c, ridx_v, c * _C, routv)
                 for c in range(n_ch)])
        pend = [fire(jobs[0][0], jobs[0][1], 0), fire(jobs[1][0], jobs[1][1], 1)]
        for k, (tbl, qslot, idx_src, off, outv) in enumerate(jobs):
            pend[k & 1].wait()
            extract(k & 1, idx_src, off, outv)
            if k + 2 < len(jobs):
                nx = jobs[k + 2]
                pend[k & 1] = fire(nx[0], nx[1], k & 1)

        pltpu.sync_copy(houtv, out_h_hbm.at[pl.ds(base * D, b_w * D)])
        pltpu.sync_copy(routv, out_r_hbm.at[pl.ds(base * D, b_w * D)])

    out_h, out_r = _gather2(heads, rels, hq, rq, et4, rt4)
    return (out_h.reshape(B, D), out_r.reshape(B, D))


# speedup vs baseline: 3.7808x; 3.7808x over previous
import functools

import jax
import jax.numpy as jnp
from jax import lax
from jax.experimental import pallas as pl
from jax.experimental.pallas import tpu as pltpu
from jax.experimental.pallas import tpu_sc as plsc

_L = 16
_NS = 4


def kernel(heads, rels, entity_table, rel_table):
    B = heads.shape[0]
    E, D = entity_table.shape
    R = rel_table.shape[0]
    NP, NC8 = D // 8, 8
    RT = R // 128
    RTAIL = R - RT * 128

    info = plsc.get_sparse_core_info()
    NW = info.num_cores * info.num_subcores
    b_w = B // NW
    n_grp = b_w // _L
    tpw = b_w // 128
    assert b_w * NW == B and n_grp * _L == b_w

    etT = entity_table.T.reshape(NP, NC8, E)
    rtT = rel_table.T

    mesh = plsc.VectorSubcoreMesh(core_axis_name="c", subcore_axis_name="s")

    @functools.partial(
        pl.kernel,
        mesh=mesh,
        compiler_params=pltpu.CompilerParams(needs_layout_passes=False),
        out_type=(
            jax.ShapeDtypeStruct((B * D,), jnp.float32),
            jax.ShapeDtypeStruct((B * D,), jnp.float32),
        ),
        scratch_types=[
            pltpu.VMEM((b_w,), jnp.int32),
            pltpu.VMEM((b_w,), jnp.int32),
            pltpu.VMEM((NP * (RT + 1), 8, 128), jnp.float32),
            pltpu.VMEM((_NS, _L * D * _L), jnp.float32),
            pltpu.VMEM((b_w * D,), jnp.float32),
            pltpu.VMEM((b_w * D,), jnp.float32),
            pltpu.SemaphoreType.DMA,
            pltpu.SemaphoreType.DMA,
            pltpu.SemaphoreType.DMA,
        ],
    )
    def _gather2(heads_hbm, rels_hbm, etT_hbm, rtT_hbm,
                 out_h_hbm, out_r_hbm,
                 hidx_v, ridx_v, relv, gbuf, houtT, routT,
                 sem_i, sem_r, sem_g):
        wid = lax.axis_index("s") * info.num_cores + lax.axis_index("c")
        base = wid * b_w
        iota = lax.iota(jnp.int32, _L)
        c8lo = iota & 7
        tp0 = (iota >> 3) * tpw
        tp1 = ((iota + _L) >> 3) * tpw

        ci = pltpu.async_copy(heads_hbm.at[pl.ds(base, b_w)], hidx_v, sem_i)
        cr = pltpu.async_copy(rels_hbm.at[pl.ds(base, b_w)], ridx_v, sem_i)

        rel_copies = []
        for i in range(NP):
            for j in range(RT):
                rel_copies.append(pltpu.async_copy(
                    rtT_hbm.at[pl.ds(8 * i, 8), pl.ds(128 * j, 128)],
                    relv.at[i * (RT + 1) + j], sem_r))
        if RTAIL:
            for c in range(D):
                rel_copies.append(pltpu.async_copy(
                    rtT_hbm.at[c, pl.ds(RT * 128, RTAIL)],
                    relv.at[(c // 8) * (RT + 1) + RT, c % 8, pl.ds(0, RTAIL)],
                    sem_r))

        ci.wait()
        cr.wait()
        for c in rel_copies:
            c.wait()

        def _fire(g):
            slot = g & (_NS - 1)
            idxv = hidx_v[pl.ds(g * _L, _L)]
            for lane in range(_L):
                r = idxv[lane]
                gg = pl.multiple_of((r >> 4) * _L, _L)
                for k in range(D):
                    pltpu.async_copy(
                        etT_hbm.at[k // 8, k % 8, pl.ds(gg, _L)],
                        gbuf.at[slot, pl.ds(lane * (D * _L) + k * _L, _L)],
                        sem_g)

        def _extract(g):
            slot = g & (_NS - 1)
            idxv = hidx_v[pl.ds(g * _L, _L)]
            rmv = idxv & 15
            slotv = iota * 0 + slot
            a16 = iota * _L
            for lane in range(_L):
                rms = iota * 0 + rmv[lane]
                addr0 = a16 + (lane * (D * _L)) + rms
                v0 = plsc.load_gather(gbuf, [slotv, addr0])
                v1 = plsc.load_gather(gbuf, [slotv, addr0 + _L * _L])
                houtT[pl.ds(g * (_L * D) + lane * D, _L)] = v0
                houtT[pl.ds(g * (_L * D) + lane * D + _L, _L)] = v1

        def _drain(g):
            pltpu.make_async_copy(
                out_h_hbm.at[pl.ds(0, _L * D * _L)],
                gbuf.at[g & (_NS - 1)], sem_g).wait()

        def _rel_group(g):
            rrv = ridx_v[pl.ds(g * _L, _L)]
            t = rrv >> 7
            m = rrv & 127
            outb = (g * _L + iota) * D
            for c2 in range(D):
                i0 = t + (c2 // 8) * (RT + 1)
                c8v = iota * 0 + (c2 % 8)
                v = plsc.load_gather(relv, [i0, c8v, m])
                plsc.store_scatter(routT, [outb + c2], v)

        for g in range(_NS - 1):
            _fire(jnp.int32(g))

        def _steady(g, _):
            _drain(g)
            _fire(g + (_NS - 1))
            _extract(g)
            _rel_group(g)
            return 0
        lax.fori_loop(0, n_grp - (_NS - 1), _steady, 0)

        def _epi(g, _):
            _drain(g)
            _extract(g)
            _rel_group(g)
            return 0
        lax.fori_loop(n_grp - (_NS - 1), n_grp, _epi, 0)

        pltpu.sync_copy(houtT, out_h_hbm.at[pl.ds(base * D, b_w * D)])
        pltpu.sync_copy(routT, out_r_hbm.at[pl.ds(base * D, b_w * D)])

    out_h, out_r = _gather2(heads, rels, etT, rtT)
    return (out_h.reshape(B, D), out_r.reshape(B, D))
